# Initial kernel scaffold; baseline (speedup 1.0000x reference)
import jax
import jax.numpy as jnp
from jax.experimental import pallas as pl


def kernel(x, edge_index, batch, W1a, b1a, W1b, b1b, eps1, W2a, b2a, W2b, b2b, eps2, Wh, bh):
    raise NotImplementedError("write your pallas kernel here")


# speedup vs baseline: 5.5257x; 5.5257x over previous
import functools

import jax
import jax.numpy as jnp
from jax import lax
from jax.experimental import pallas as pl
from jax.experimental.pallas import tpu as pltpu
from jax.experimental.pallas import tpu_sc as plsc

N = 10000
E = 320000
D = 128
H = 64
G = 128

NC = 2
NS = 16
NW = NC * NS

SINK = N
ROWS_PT = 632
N_PAD = NS * ROWS_PT


def _make_segsum(width, CHUNK, NCH):
    def body(table, srcix, dstix, zeros, out, src_v, dst_v, rows_v, acc_sh,
             sem):
        c = lax.axis_index("c")
        s = lax.axis_index("s")
        wid = s * NC + c
        r0 = s * ROWS_PT
        pltpu.sync_copy(zeros.at[pl.ds(r0, ROWS_PT)],
                        acc_sh.at[pl.ds(r0, ROWS_PT)])
        pltpu.sync_copy(srcix.at[wid], src_v)
        pltpu.sync_copy(dstix.at[wid], dst_v)
        plsc.subcore_barrier()

        pltpu.async_copy(table.at[src_v.at[0]], rows_v.at[0], sem)

        def chunk(j, carry):
            b = j % 2
            pltpu.make_async_copy(table.at[src_v.at[j]], rows_v.at[b],
                                  sem).wait()

            @pl.when(j + 1 < NCH)
            def _():
                pltpu.async_copy(table.at[src_v.at[j + 1]], rows_v.at[1 - b],
                                 sem)

            pltpu.sync_copy(rows_v.at[b], acc_sh.at[dst_v.at[j]], add=True)
            return carry

        lax.fori_loop(0, NCH, chunk, 0)
        plsc.subcore_barrier()
        pltpu.sync_copy(acc_sh.at[pl.ds(r0, ROWS_PT)],
                        out.at[pl.ds(c * N_PAD + r0, ROWS_PT)])

    return pl.kernel(
        body,
        out_type=jax.ShapeDtypeStruct((NC * N_PAD, width), jnp.float32),
        mesh=plsc.VectorSubcoreMesh(core_axis_name="c", subcore_axis_name="s"),
        compiler_params=pltpu.CompilerParams(use_tc_tiling_on_sc=False),
        scratch_types=[
            pltpu.VMEM((NCH, CHUNK), jnp.int32),
            pltpu.VMEM((NCH, CHUNK), jnp.int32),
            pltpu.VMEM((2, CHUNK, width), jnp.float32),
            pltpu.VMEM_SHARED((N_PAD, width), jnp.float32),
            pltpu.SemaphoreType.DMA,
        ],
    )


CHUNK_D, NCH_D = 64, 158
CHUNK_H, NCH_H = 128, 80
_segsum_d = _make_segsum(D, CHUNK_D, NCH_D)
_segsum_h = _make_segsum(H, CHUNK_H, NCH_H)


def _layer1_body(x_ref, parts_ref, eps_ref, w1a_ref, b1a_ref, w1b_ref,
                 b1b_ref, h1_ref):
    agg = parts_ref[0:N, :] + parts_ref[N_PAD:N_PAD + N, :]
    t = (1.0 + eps_ref[0, 0]) * x_ref[...] + agg
    m = jnp.maximum(jnp.dot(t, w1a_ref[...],
                            preferred_element_type=jnp.float32)
                    + b1a_ref[...], 0.0)
    h = jnp.dot(m, w1b_ref[...],
                preferred_element_type=jnp.float32) + b1b_ref[...]
    h1_ref[...] = jnp.maximum(h, 0.0)


def _final_body(h1_ref, parts_ref, eps_ref, w2a_ref, b2a_ref, w2b_ref,
                b2b_ref, batch_ref, wh_ref, bh_ref, hc_ref, pred_ref):
    agg = parts_ref[0:N, :] + parts_ref[N_PAD:N_PAD + N, :]
    t = (1.0 + eps_ref[0, 0]) * h1_ref[...] + agg
    m = jnp.maximum(jnp.dot(t, w2a_ref[...],
                            preferred_element_type=jnp.float32)
                    + b2a_ref[...], 0.0)
    h2 = jnp.dot(m, w2b_ref[...],
                 preferred_element_type=jnp.float32) + b2b_ref[...]
    gids = lax.broadcasted_iota(jnp.int32, (G, N), 0)
    oh = (gids == batch_ref[...]).astype(jnp.float32)
    sums = jnp.dot(oh, h2, preferred_element_type=jnp.float32,
                   precision=lax.Precision.HIGHEST)
    counts = jnp.sum(oh, axis=1, keepdims=True)
    hc = sums / jnp.maximum(counts, 1.0)
    hc_ref[...] = hc
    pred_ref[...] = jnp.dot(hc, wh_ref[...],
                            preferred_element_type=jnp.float32) + bh_ref[...]


def kernel(x, edge_index, batch, W1a, b1a, W1b, b1b, eps1, W2a, b2a, W2b, b2b,
           eps2, Wh, bh):
    f32 = jnp.float32
    src = edge_index[0]
    dst = edge_index[1]
    def mk_idx(v, fill, nch, chunk):
        pad = NW * nch * chunk - E
        vp = jnp.concatenate([v, jnp.full((pad,), fill, jnp.int32)])
        return vp.reshape(NW, nch, chunk)

    srcix_d = mk_idx(src, 0, NCH_D, CHUNK_D)
    dstix_d = mk_idx(dst, SINK, NCH_D, CHUNK_D)
    srcix_h = mk_idx(src, 0, NCH_H, CHUNK_H)
    dstix_h = mk_idx(dst, SINK, NCH_H, CHUNK_H)
    zeros_d = jnp.zeros((N_PAD, D), f32)
    zeros_h = jnp.zeros((N_PAD, H), f32)

    parts1 = _segsum_d(x, srcix_d, dstix_d, zeros_d)

    smem_spec = pl.BlockSpec(memory_space=pltpu.SMEM)

    h1 = pl.pallas_call(
        _layer1_body,
        out_shape=jax.ShapeDtypeStruct((N, H), f32),
        in_specs=[pl.BlockSpec(), pl.BlockSpec(), smem_spec, pl.BlockSpec(),
                  pl.BlockSpec(), pl.BlockSpec(), pl.BlockSpec()],
    )(x, parts1, jnp.reshape(eps1, (1, 1)), W1a, b1a.reshape(1, H), W1b,
      b1b.reshape(1, H))

    parts2 = _segsum_h(h1, srcix_h, dstix_h, zeros_h)

    hc, pred = pl.pallas_call(
        _final_body,
        out_shape=(jax.ShapeDtypeStruct((G, H), f32),
                   jax.ShapeDtypeStruct((G, 1), f32)),
        in_specs=[pl.BlockSpec(), pl.BlockSpec(), smem_spec, pl.BlockSpec(),
                  pl.BlockSpec(), pl.BlockSpec(), pl.BlockSpec(),
                  pl.BlockSpec(), pl.BlockSpec(), pl.BlockSpec()],
    )(h1, parts2, jnp.reshape(eps2, (1, 1)), W2a, b2a.reshape(1, H), W2b,
      b2b.reshape(1, H), batch.reshape(1, N), Wh, bh.reshape(1, 1))

    return (hc, pred)


# speedup vs baseline: 15.0262x; 2.7193x over previous
import jax
import jax.numpy as jnp
from jax import lax
from jax.experimental import pallas as pl
from jax.experimental.pallas import tpu as pltpu
from jax.experimental.pallas import tpu_sc as plsc

N = 10000
E = 320000
D = 128
H = 64
G = 128

NC = 2
NS = 16
NW = NC * NS

ROWS_PT = 640
R_STEP = 624
N_PAD = 15 * R_STEP + ROWS_PT
CHUNK = 80
NCH = 125


def _make_segsum(width, tc_tiling):

    def body(table, srcix, dstix, out, src_v, dst_v, rows_v, acc_sh, sem):
        c = lax.axis_index("c")
        s = lax.axis_index("s")
        wid = c * NS + s
        r0 = s * R_STEP
        icp1 = pltpu.async_copy(srcix.at[wid], src_v, sem)
        icp2 = pltpu.async_copy(dstix.at[wid], dst_v, sem)

        zvec = jnp.zeros((16,), jnp.float32)

        def zv(i, carry):
            rr = i // (width // 16)
            kk = i % (width // 16)
            rows_v[0, rr, pl.ds(kk * 16, 16)] = zvec
            return carry

        lax.fori_loop(0, CHUNK * width // 16, zv, 0)

        def zc(k, carry):
            pltpu.sync_copy(rows_v.at[0],
                            acc_sh.at[pl.ds(r0 + k * CHUNK, CHUNK)])
            return carry

        lax.fori_loop(0, ROWS_PT // CHUNK, zc, 0)
        icp1.wait()
        icp2.wait()
        plsc.subcore_barrier()

        pltpu.async_copy(table.at[src_v.at[0]], rows_v.at[0], sem)
        pltpu.async_copy(table.at[src_v.at[1]], rows_v.at[1], sem)

        def chunk(j, carry):
            b = j % 3
            pltpu.make_async_copy(table.at[src_v.at[j]], rows_v.at[b],
                                  sem).wait()

            @pl.when(j + 2 < NCH)
            def _():
                pltpu.async_copy(table.at[src_v.at[j + 2]],
                                 rows_v.at[(j + 2) % 3], sem)

            pltpu.sync_copy(rows_v.at[b], acc_sh.at[dst_v.at[j]], add=True)
            return carry

        lax.fori_loop(0, NCH, chunk, 0)
        plsc.subcore_barrier()
        pltpu.sync_copy(acc_sh.at[pl.ds(r0, ROWS_PT)],
                        out.at[pl.ds(c * N_PAD + r0, ROWS_PT)])

    return pl.kernel(
        body,
        out_type=jax.ShapeDtypeStruct((NC * N_PAD, width), jnp.float32),
        mesh=plsc.VectorSubcoreMesh(core_axis_name="c", subcore_axis_name="s"),
        compiler_params=pltpu.CompilerParams(use_tc_tiling_on_sc=tc_tiling),
        scratch_types=[
            pltpu.VMEM((NCH, CHUNK), jnp.int32),
            pltpu.VMEM((NCH, CHUNK), jnp.int32),
            pltpu.VMEM((3, CHUNK, width), jnp.float32),
            pltpu.VMEM_SHARED((N_PAD, width), jnp.float32),
            pltpu.SemaphoreType.DMA,
        ],
    )


_segsum_d = _make_segsum(D, False)
_segsum_h = _make_segsum(H, False)


def _layer1_body(x_ref, parts_ref, eps_ref, w1a_ref, b1a_ref, w1b_ref,
                 b1b_ref, h1_ref):
    agg = parts_ref[0:N, :] + parts_ref[N_PAD:N_PAD + N, :]
    t = (1.0 + eps_ref[0, 0]) * x_ref[...] + agg
    m = jnp.maximum(jnp.dot(t, w1a_ref[...],
                            preferred_element_type=jnp.float32)
                    + b1a_ref[...], 0.0)
    h = jnp.dot(m, w1b_ref[...],
                preferred_element_type=jnp.float32) + b1b_ref[...]
    h1_ref[...] = jnp.maximum(h, 0.0)


def _final_body(h1_ref, parts_ref, eps_ref, w2a_ref, b2a_ref, w2b_ref,
                b2b_ref, batch_ref, wh_ref, bh_ref, hc_ref, pred_ref):
    agg = parts_ref[0:N, :] + parts_ref[N_PAD:N_PAD + N, :]
    t = (1.0 + eps_ref[0, 0]) * h1_ref[...] + agg
    m = jnp.maximum(jnp.dot(t, w2a_ref[...],
                            preferred_element_type=jnp.float32)
                    + b2a_ref[...], 0.0)
    h2 = jnp.dot(m, w2b_ref[...],
                 preferred_element_type=jnp.float32) + b2b_ref[...]
    gids = lax.broadcasted_iota(jnp.int32, (G, N), 0)
    oh = (gids == batch_ref[...]).astype(jnp.float32)
    sums = jnp.dot(oh, h2, preferred_element_type=jnp.float32,
                   precision=lax.Precision.HIGHEST)
    counts = jnp.sum(oh, axis=1, keepdims=True)
    hc = sums / jnp.maximum(counts, 1.0)
    hc_ref[...] = hc
    pred_ref[...] = jnp.dot(hc, wh_ref[...],
                            preferred_element_type=jnp.float32) + bh_ref[...]


def kernel(x, edge_index, batch, W1a, b1a, W1b, b1b, eps1, W2a, b2a, W2b, b2b,
           eps2, Wh, bh):
    f32 = jnp.float32
    src = edge_index[0]
    dst = edge_index[1]
    srcix = src.reshape(NW, NCH, CHUNK)
    dstix = dst.reshape(NW, NCH, CHUNK)
    parts1 = _segsum_d(x, srcix, dstix)

    smem_spec = pl.BlockSpec(memory_space=pltpu.SMEM)

    h1 = pl.pallas_call(
        _layer1_body,
        out_shape=jax.ShapeDtypeStruct((N, H), f32),
        in_specs=[pl.BlockSpec(), pl.BlockSpec(), smem_spec, pl.BlockSpec(),
                  pl.BlockSpec(), pl.BlockSpec(), pl.BlockSpec()],
    )(x, parts1, jnp.reshape(eps1, (1, 1)), W1a, b1a.reshape(1, H), W1b,
      b1b.reshape(1, H))

    parts2 = _segsum_h(h1, srcix, dstix)

    hc, pred = pl.pallas_call(
        _final_body,
        out_shape=(jax.ShapeDtypeStruct((G, H), f32),
                   jax.ShapeDtypeStruct((G, 1), f32)),
        in_specs=[pl.BlockSpec(), pl.BlockSpec(), smem_spec, pl.BlockSpec(),
                  pl.BlockSpec(), pl.BlockSpec(), pl.BlockSpec(),
                  pl.BlockSpec(), pl.BlockSpec(), pl.BlockSpec()],
    )(h1, parts2, jnp.reshape(eps2, (1, 1)), W2a, b2a.reshape(1, H), W2b,
      b2b.reshape(1, H), batch.reshape(1, N), Wh, bh.reshape(1, 1))

    return (hc, pred)
